# baseline (speedup 1.0000x reference)
import functools

import jax
import jax.numpy as jnp
import numpy as np
from jax import lax
from jax.experimental import pallas as pl
from jax.experimental.pallas import tpu as pltpu
from jax.experimental.pallas import tpu_sc as plsc

_LO = float(np.log(0.001 / 0.999))
_HI = -_LO

_SPW = 16


def _tc_body(cm_ref, cb_ref, ctx_ref, w_ref, lg_ref, p_ref, off_ref,
             *, sb, m, nb):
    nbit = sb * m
    nrow = sb * nb
    f32 = jnp.float32
    b = ctx_ref.shape[0]

    cm = cm_ref[...].reshape(nbit, cm_ref.shape[-1])
    d = lax.dot_general(cm, ctx_ref[...], (((1,), (1,)), ((), ())),
                        preferred_element_type=f32)
    cb = cb_ref[...].reshape(nbit, 1)
    bits = (d > cb).astype(f32)
    r = lax.broadcasted_iota(jnp.int32, (sb, nbit), 1)
    s = lax.broadcasted_iota(jnp.int32, (sb, nbit), 0)
    a4 = jnp.where(r // m == s, lax.shift_left(1, r % m).astype(f32), 0.0)
    idx = jnp.dot(a4, bits, preferred_element_type=f32)

    sg = (pl.program_id(0) * sb
          + lax.broadcasted_iota(jnp.int32, (sb, b), 0))
    bcol = lax.broadcasted_iota(jnp.int32, (sb, b), 1)
    wn = (sg + 1) // _SPW
    cs = sg - jnp.maximum(_SPW * wn - 1, 0)
    off_ref[...] = (cs * nb + idx.astype(jnp.int32)) * b + bcol

    w = w_ref[...].reshape(nrow, w_ref.shape[-1])
    p_ref[...] = lax.dot_general(w, lg_ref[...], (((1,), (1,)), ((), ())),
                                 preferred_element_type=f32)


def _sc_select(p_hbm, off_hbm, bias_hbm, out_hbm, chunk, offs, outbuf,
               bias_v):
    wid = lax.axis_index("s") * 2 + lax.axis_index("c")
    b = outbuf.shape[0]
    nel = _SPW * 16 * b
    base_p = pl.multiple_of(jnp.maximum(wid * nel - 16 * b, 0), 1024)
    base_o = pl.multiple_of(jnp.maximum(wid * _SPW * b - b, 0), 64)
    pltpu.sync_copy(p_hbm.at[pl.ds(base_p, nel)], chunk)
    pltpu.sync_copy(off_hbm.at[pl.ds(base_o, _SPW * b)], offs)
    pltpu.sync_copy(bias_hbm, bias_v)
    bv = bias_v[...]
    lane = lax.iota(jnp.int32, 16)
    shift = jnp.where(wid == 0, b, 0)
    is_bias = jnp.logical_and(wid == 0, lane == 0)
    for bi in range(b):
        loc = jnp.maximum(lane * b + (bi - shift), 0)
        off = plsc.load_gather(offs, [loc])
        val = plsc.load_gather(chunk, [off])
        val = jnp.minimum(jnp.maximum(val, _LO), _HI)
        outbuf[bi, pl.ds(0, 16)] = jnp.where(is_bias, bv, val)
    pltpu.sync_copy(outbuf, out_hbm.at[:, pl.ds(wid * _SPW, _SPW)])


def kernel(logit, context, context_maps, context_bias, weights, bias,
           boolean_converter):
    B, I = logit.shape
    _, C = context.shape
    K, S, M, _ = context_maps.shape
    NB = weights.shape[2]
    N = K * S
    SB = 32
    G = -(-N // SB)
    NP = G * SB

    cm = context_maps.reshape(N, M, C)
    cb = context_bias.reshape(N, M, 1)
    wt = weights.reshape(N, NB, I)

    p, off = pl.pallas_call(
        functools.partial(_tc_body, sb=SB, m=M, nb=NB),
        grid=(G,),
        in_specs=[
            pl.BlockSpec((SB, M, C), lambda i: (i, 0, 0)),
            pl.BlockSpec((SB, M, 1), lambda i: (i, 0, 0)),
            pl.BlockSpec((B, C), lambda i: (0, 0)),
            pl.BlockSpec((SB, NB, I), lambda i: (i, 0, 0)),
            pl.BlockSpec((B, I), lambda i: (0, 0)),
        ],
        out_specs=[
            pl.BlockSpec((SB * NB, B), lambda i: (i, 0)),
            pl.BlockSpec((SB, B), lambda i: (i, 0)),
        ],
        out_shape=[
            jax.ShapeDtypeStruct((NP * NB, B), jnp.float32),
            jax.ShapeDtypeStruct((NP, B), jnp.int32),
        ],
    )(cm, cb, context, wt, logit)

    bias_rep = jnp.broadcast_to(bias.reshape(K), (16,))

    out2d = pl.kernel(
        _sc_select,
        out_type=jax.ShapeDtypeStruct((B, N + 1), jnp.float32),
        mesh=plsc.VectorSubcoreMesh(core_axis_name="c", subcore_axis_name="s"),
        compiler_params=pltpu.CompilerParams(needs_layout_passes=False,
                                             use_tc_tiling_on_sc=False),
        scratch_types=[
            pltpu.VMEM((_SPW * NB * B,), jnp.float32),
            pltpu.VMEM((_SPW * B,), jnp.int32),
            pltpu.VMEM((B, _SPW), jnp.float32),
            pltpu.VMEM((16,), jnp.float32),
        ],
    )(p.reshape(NP * NB * B), off.reshape(NP * B), bias_rep)

    return out2d.reshape(B, N + 1, K)


# speedup vs baseline: 1.2319x; 1.2319x over previous
import functools

import jax
import jax.numpy as jnp
import numpy as np
from jax import lax
from jax.experimental import pallas as pl
from jax.experimental.pallas import tpu as pltpu

_LO = float(np.log(0.001 / 0.999))
_HI = -_LO


def _body(cm_ref, cb_ref, ctx_ref, w_ref, lg_ref, bias_ref, out_ref,
          carry_ref, *, sb, m, nb, n):
    nbit = sb * m
    nrow = sb * nb
    f32 = jnp.float32
    b = ctx_ref.shape[0]
    i = pl.program_id(0)

    cm = cm_ref[...].reshape(nbit, cm_ref.shape[-1])
    d = lax.dot_general(cm, ctx_ref[...], (((1,), (1,)), ((), ())),
                        preferred_element_type=f32)
    cb = cb_ref[...].reshape(nbit, 1)
    bits = (d > cb).astype(f32)
    r = lax.broadcasted_iota(jnp.int32, (sb, nbit), 1)
    s = lax.broadcasted_iota(jnp.int32, (sb, nbit), 0)
    a4 = jnp.where(r // m == s, lax.shift_left(1, r % m).astype(f32), 0.0)
    idx = jnp.dot(a4, bits, preferred_element_type=f32)

    w = w_ref[...].reshape(nrow, w_ref.shape[-1])
    p = lax.dot_general(w, lg_ref[...], (((1,), (1,)), ((), ())),
                        preferred_element_type=f32)

    rr = lax.broadcasted_iota(jnp.int32, (nrow, sb), 0)
    ss = lax.broadcasted_iota(jnp.int32, (nrow, sb), 1)
    e_exp = (rr // nb == ss).astype(f32)
    idx_exp = lax.dot_general(e_exp, idx, (((1,), (0,)), ((), ())),
                              preferred_element_type=f32)
    riota = lax.broadcasted_iota(jnp.int32, (nrow, b), 0)
    rbucket = (riota % nb).astype(f32)
    nv = n - i * sb
    valid = (riota // nb) < nv
    masked = jnp.where(jnp.logical_and(idx_exp == rbucket, valid), p, 0.0)

    e_shift = (rr // nb == ss - 1).astype(f32)
    virt = lax.dot_general(e_shift, masked, (((0,), (0,)), ((), ())),
                           preferred_element_type=f32)
    virt = jnp.clip(virt, _LO, _HI)

    first = jnp.where(i == 0, bias_ref[0:1, 0:b], carry_ref[0:1, 0:b])
    rowi = lax.broadcasted_iota(jnp.int32, (sb, b), 0)
    rows = jnp.where(rowi == 0, jnp.broadcast_to(first, (sb, b)), virt)
    out_ref[...] = rows.T

    last = jnp.sum(masked[nrow - nb:, :], axis=0, keepdims=True)
    carry_ref[0:1, 0:b] = jnp.clip(last, _LO, _HI)


def kernel(logit, context, context_maps, context_bias, weights, bias,
           boolean_converter):
    B, I = logit.shape
    _, C = context.shape
    K, S, M, _ = context_maps.shape
    NB = weights.shape[2]
    N = K * S
    SB = 128
    G = -(-N // SB)

    cm = context_maps.reshape(N, M, C)
    cb = context_bias.reshape(N, M, 1)
    wt = weights.reshape(N, NB, I)
    bias2d = jnp.broadcast_to(bias.reshape(1, 1), (8, B))

    out2d = pl.pallas_call(
        functools.partial(_body, sb=SB, m=M, nb=NB, n=N),
        grid=(G,),
        in_specs=[
            pl.BlockSpec((SB, M, C), lambda i: (i, 0, 0)),
            pl.BlockSpec((SB, M, 1), lambda i: (i, 0, 0)),
            pl.BlockSpec((B, C), lambda i: (0, 0)),
            pl.BlockSpec((SB, NB, I), lambda i: (i, 0, 0)),
            pl.BlockSpec((B, I), lambda i: (0, 0)),
            pl.BlockSpec((8, B), lambda i: (0, 0)),
        ],
        out_specs=pl.BlockSpec((B, SB), lambda i: (0, i)),
        out_shape=jax.ShapeDtypeStruct((B, G * SB), jnp.float32),
        scratch_shapes=[pltpu.VMEM((8, B), jnp.float32)],
    )(cm, cb, context, wt, logit, bias2d)

    return out2d[:, :N + 1].reshape(B, N + 1, K)


# speedup vs baseline: 4.6843x; 3.8025x over previous
import functools

import jax
import jax.numpy as jnp
import numpy as np
from jax import lax
from jax.experimental import pallas as pl
from jax.experimental.pallas import tpu as pltpu

_LO = float(np.log(0.001 / 0.999))
_HI = -_LO


def _body(cmt_ref, cbt_ref, ctx_ref, w_ref, lg_ref, bias_ref, out_ref,
          carry_ref, *, sb, m, nb, n):
    nrow = sb * nb
    f32 = jnp.float32
    b = ctx_ref.shape[0]
    i = pl.program_id(0)

    ctx = ctx_ref[...]
    idx = jnp.zeros((b, sb), f32)
    for j in range(m):
        dj = jnp.dot(ctx, cmt_ref[0, j], preferred_element_type=f32)
        bitj = (dj > cbt_ref[0, j]).astype(f32)
        idx = idx + float(2 ** j) * bitj

    w = w_ref[...].reshape(nrow, w_ref.shape[-1])
    p = lax.dot_general(lg_ref[...], w, (((1,), (1,)), ((), ())),
                        preferred_element_type=f32)

    rr = lax.broadcasted_iota(jnp.int32, (sb, nrow), 1)
    ss = lax.broadcasted_iota(jnp.int32, (sb, nrow), 0)
    e_exp = (rr // nb == ss).astype(f32)
    idx_exp = jnp.dot(idx, e_exp, preferred_element_type=f32)
    liota = lax.broadcasted_iota(jnp.int32, (b, nrow), 1)
    lbucket = (liota % nb).astype(f32)
    nv = n - i * sb
    valid = (liota // nb) < nv
    masked = jnp.where(jnp.logical_and(idx_exp == lbucket, valid), p, 0.0)

    rr2 = lax.broadcasted_iota(jnp.int32, (nrow, sb), 0)
    ss2 = lax.broadcasted_iota(jnp.int32, (nrow, sb), 1)
    e_shift = (rr2 // nb == ss2 - 1).astype(f32)
    virt = jnp.dot(masked, e_shift, preferred_element_type=f32)
    virt = jnp.clip(virt, _LO, _HI)

    first = jnp.where(i == 0,
                      jnp.broadcast_to(bias_ref[0:1, 0:1], (b, 1)),
                      carry_ref[0:b, 0:1])
    lane = lax.broadcasted_iota(jnp.int32, (b, sb), 1)
    out_ref[...] = jnp.where(lane == 0, jnp.broadcast_to(first, (b, sb)),
                             virt)

    last = jnp.sum(masked[:, nrow - nb:], axis=1, keepdims=True)
    carry_ref[0:b, 0:1] = jnp.clip(last, _LO, _HI)


def kernel(logit, context, context_maps, context_bias, weights, bias,
           boolean_converter):
    B, I = logit.shape
    _, C = context.shape
    K, S, M, _ = context_maps.shape
    NB = weights.shape[2]
    N = K * S
    SB = 128
    G = -(-N // SB)

    cmt = jnp.transpose(context_maps, (0, 2, 3, 1))
    cbt = jnp.transpose(context_bias, (0, 2, 3, 1))
    wt = weights.reshape(N, NB, I)
    bias2d = jnp.broadcast_to(bias.reshape(1, 1), (8, B))

    out2d = pl.pallas_call(
        functools.partial(_body, sb=SB, m=M, nb=NB, n=N),
        grid=(G,),
        in_specs=[
            pl.BlockSpec((1, M, C, SB), lambda i: (0, 0, 0, i)),
            pl.BlockSpec((1, M, 1, SB), lambda i: (0, 0, 0, i)),
            pl.BlockSpec((B, C), lambda i: (0, 0)),
            pl.BlockSpec((SB, NB, I), lambda i: (i, 0, 0)),
            pl.BlockSpec((B, I), lambda i: (0, 0)),
            pl.BlockSpec((8, B), lambda i: (0, 0)),
        ],
        out_specs=pl.BlockSpec((B, SB), lambda i: (0, i)),
        out_shape=jax.ShapeDtypeStruct((B, G * SB), jnp.float32),
        scratch_shapes=[pltpu.VMEM((B, 128), jnp.float32)],
    )(pltpu.with_memory_space_constraint(cmt, pltpu.MemorySpace.HBM),
      cbt, context, wt, logit, bias2d)

    return out2d[:, :N + 1].reshape(B, N + 1, K)
